# baseline (speedup 1.0000x reference)
import functools

import jax
import jax.numpy as jnp
from jax import lax
from jax.experimental import pallas as pl
from jax.experimental.pallas import tpu as pltpu
from jax.experimental.pallas import tpu_sc as plsc

BATCH = 16384
EMB_DIM = 32
HIDDEN = 128

_NC = 2
_NS = 16
_NW = _NC * _NS
_B_PER_W = BATCH // _NW
_CHUNK = 128
_NCHUNK = _B_PER_W // _CHUNK


_K = 16
_NGROUP = _B_PER_W // _K


def _gather_body(table_hbm, idx_hbm, out_hbm, idx_v, sem):
    wid = lax.axis_index("s") * _NC + lax.axis_index("c")
    base = wid * _B_PER_W
    pltpu.sync_copy(idx_hbm.at[wid], idx_v)

    def group(g, carry):
        r0 = g * _K
        vec = idx_v[pl.ds(r0, _K)]
        copies = []
        for j in range(_K):
            u = vec[j]
            copies.append(
                pltpu.async_copy(
                    table_hbm.at[pl.ds(u, 1)],
                    out_hbm.at[pl.ds(base + r0 + j, 1)],
                    sem,
                )
            )
        for c in copies:
            c.wait()
        return carry

    lax.fori_loop(0, _NGROUP, group, 0)


@functools.partial(jax.jit, static_argnames=())
def _sc_gather(table, idx):
    mesh = plsc.VectorSubcoreMesh(core_axis_name="c", subcore_axis_name="s")
    k = pl.kernel(
        _gather_body,
        out_type=jax.ShapeDtypeStruct((BATCH, EMB_DIM), jnp.float32),
        mesh=mesh,
        scratch_types=[
            pltpu.VMEM((_B_PER_W,), jnp.int32),
            pltpu.SemaphoreType.DMA,
        ],
    )
    return k(table, idx.reshape(_NW, _B_PER_W))


def _dense_body(emb_ref, w_ref, b_ref, g_ref, beta_ref, out_ref):
    h = jnp.dot(emb_ref[...], w_ref[...], preferred_element_type=jnp.float32)
    h = h + b_ref[...]
    h = jnp.maximum(h, 0.0)
    mean = jnp.mean(h, axis=-1, keepdims=True)
    var = jnp.mean((h - mean) ** 2, axis=-1, keepdims=True)
    h = (h - mean) * lax.rsqrt(var + 1e-5)
    out_ref[...] = h * g_ref[...] + beta_ref[...]


_BLK = 2048


def _tc_dense(emb, W, b, gamma, beta):
    grid = (BATCH // _BLK,)
    return pl.pallas_call(
        _dense_body,
        grid=grid,
        in_specs=[
            pl.BlockSpec((_BLK, EMB_DIM), lambda i: (i, 0)),
            pl.BlockSpec((EMB_DIM, HIDDEN), lambda i: (0, 0)),
            pl.BlockSpec((1, HIDDEN), lambda i: (0, 0)),
            pl.BlockSpec((1, HIDDEN), lambda i: (0, 0)),
            pl.BlockSpec((1, HIDDEN), lambda i: (0, 0)),
        ],
        out_specs=pl.BlockSpec((_BLK, HIDDEN), lambda i: (i, 0)),
        out_shape=jax.ShapeDtypeStruct((BATCH, HIDDEN), jnp.float32),
    )(emb, W, b.reshape(1, HIDDEN), gamma.reshape(1, HIDDEN),
      beta.reshape(1, HIDDEN))


def kernel(user_ids, table, W, b, gamma, beta):
    emb = _sc_gather(table, user_ids.astype(jnp.int32))
    return _tc_dense(emb, W, b, gamma, beta)


# speedup vs baseline: 1.0695x; 1.0695x over previous
import functools

import jax
import jax.numpy as jnp
from jax import lax
from jax.experimental import pallas as pl
from jax.experimental.pallas import tpu as pltpu
from jax.experimental.pallas import tpu_sc as plsc

BATCH = 16384
EMB_DIM = 32
HIDDEN = 128

_NC = 2
_NS = 16
_NW = _NC * _NS
_B_PER_W = BATCH // _NW
_CHUNK = 128
_NCHUNK = _B_PER_W // _CHUNK
_GRP = 16


def _gather_body(table_hbm, rows_hbm, idxf_hbm, out_hbm,
                 ridx_v, idx_f, rows_v, emb_v, sem):
    wid = lax.axis_index("s") * _NC + lax.axis_index("c")
    base = wid * _B_PER_W
    pltpu.sync_copy(rows_hbm.at[wid], ridx_v)
    pltpu.sync_copy(idxf_hbm.at[wid], idx_f)
    lane_iota = lax.iota(jnp.int32, 16)
    for h in range(2):
        copies = [
            pltpu.async_copy(
                table_hbm.at[ridx_v.at[2 * h + j]],
                rows_v.at[pl.ds(j * _CHUNK, _CHUNK)],
                sem,
            )
            for j in range(_NCHUNK // 2)
        ]
        for c in copies:
            c.wait()

        def group(g, carry):
            r0 = g * _GRP
            vec = idx_f[pl.ds(h * 256 + r0, _GRP)]
            for j in range(_GRP):
                off = (vec[j] & 3) * 32
                row = jnp.full((16,), r0 + j, jnp.int32)
                erow = jnp.full((16,), h * 256 + r0 + j, jnp.int32)
                for k in range(2):
                    lanes = off + k * 16 + lane_iota
                    val = plsc.load_gather(rows_v, [row, lanes])
                    plsc.store_scatter(emb_v, [erow, k * 16 + lane_iota], val)
            return carry

        lax.fori_loop(0, 256 // _GRP, group, 0)
    pltpu.sync_copy(emb_v, out_hbm.at[pl.ds(base, _B_PER_W)])


@functools.partial(jax.jit, static_argnames=())
def _sc_gather(table128, rows, idxf):
    mesh = plsc.VectorSubcoreMesh(core_axis_name="c", subcore_axis_name="s")
    k = pl.kernel(
        _gather_body,
        out_type=jax.ShapeDtypeStruct((BATCH, EMB_DIM), jnp.float32),
        mesh=mesh,
        scratch_types=[
            pltpu.VMEM((_NCHUNK, _CHUNK), jnp.int32),
            pltpu.VMEM((_B_PER_W,), jnp.int32),
            pltpu.VMEM((_B_PER_W // 2, 4 * EMB_DIM), jnp.float32),
            pltpu.VMEM((_B_PER_W, EMB_DIM), jnp.float32),
            pltpu.SemaphoreType.DMA,
        ],
        compiler_params=pltpu.CompilerParams(needs_layout_passes=False),
    )
    return k(table128, rows.reshape(_NW, _NCHUNK, _CHUNK),
             idxf.reshape(_NW, _B_PER_W))


def _dense_body(emb_ref, w_ref, b_ref, g_ref, beta_ref, out_ref):
    h = jnp.dot(emb_ref[...], w_ref[...], preferred_element_type=jnp.float32)
    h = h + b_ref[...]
    h = jnp.maximum(h, 0.0)
    mean = jnp.mean(h, axis=-1, keepdims=True)
    var = jnp.mean((h - mean) ** 2, axis=-1, keepdims=True)
    h = (h - mean) * lax.rsqrt(var + 1e-5)
    out_ref[...] = h * g_ref[...] + beta_ref[...]


_BLK = 2048


def _tc_dense(emb, W, b, gamma, beta):
    grid = (BATCH // _BLK,)
    return pl.pallas_call(
        _dense_body,
        grid=grid,
        in_specs=[
            pl.BlockSpec((_BLK, EMB_DIM), lambda i: (i, 0)),
            pl.BlockSpec((EMB_DIM, HIDDEN), lambda i: (0, 0)),
            pl.BlockSpec((1, HIDDEN), lambda i: (0, 0)),
            pl.BlockSpec((1, HIDDEN), lambda i: (0, 0)),
            pl.BlockSpec((1, HIDDEN), lambda i: (0, 0)),
        ],
        out_specs=pl.BlockSpec((_BLK, HIDDEN), lambda i: (i, 0)),
        out_shape=jax.ShapeDtypeStruct((BATCH, HIDDEN), jnp.float32),
    )(emb, W, b.reshape(1, HIDDEN), gamma.reshape(1, HIDDEN),
      beta.reshape(1, HIDDEN))


def kernel(user_ids, table, W, b, gamma, beta):
    ids = user_ids.astype(jnp.int32)
    table128 = jnp.reshape(table, (table.shape[0] // 4, 4 * EMB_DIM))
    emb = _sc_gather(table128, ids >> 2, ids)
    return _tc_dense(emb, W, b, gamma, beta)


# speedup vs baseline: 1.1074x; 1.0355x over previous
import functools

import jax
import jax.numpy as jnp
from jax import lax
from jax.experimental import pallas as pl
from jax.experimental.pallas import tpu as pltpu
from jax.experimental.pallas import tpu_sc as plsc

BATCH = 16384
EMB_DIM = 32
HIDDEN = 128
PAD_DIM = 128

_NC = 2
_NS = 16
_NW = _NC * _NS
_B_PER_W = BATCH // _NW
_CHUNK = 128
_NCHUNK = _B_PER_W // _CHUNK


def _gather_body(table_hbm, idx_hbm, out_hbm, idx_v, rows_v, sem):
    wid = lax.axis_index("s") * _NC + lax.axis_index("c")
    base = wid * _B_PER_W
    pltpu.sync_copy(idx_hbm.at[wid], idx_v)
    copies = [
        pltpu.async_copy(
            table_hbm.at[idx_v.at[j]],
            rows_v.at[pl.ds(j * _CHUNK, _CHUNK)],
            sem,
        )
        for j in range(_NCHUNK)
    ]
    for c in copies:
        c.wait()
    pltpu.sync_copy(rows_v, out_hbm.at[pl.ds(base, _B_PER_W)])


@functools.partial(jax.jit, static_argnames=())
def _sc_gather(tablep, idx):
    mesh = plsc.VectorSubcoreMesh(core_axis_name="c", subcore_axis_name="s")
    k = pl.kernel(
        _gather_body,
        out_type=jax.ShapeDtypeStruct((BATCH, PAD_DIM), jnp.float32),
        mesh=mesh,
        scratch_types=[
            pltpu.VMEM((_NCHUNK, _CHUNK), jnp.int32),
            pltpu.VMEM((_B_PER_W, PAD_DIM), jnp.float32),
            pltpu.SemaphoreType.DMA,
        ],
    )
    return k(tablep, idx.reshape(_NW, _NCHUNK, _CHUNK))


def _dense_body(emb_ref, w_ref, b_ref, g_ref, beta_ref, out_ref):
    h = jnp.dot(emb_ref[...], w_ref[...], preferred_element_type=jnp.float32)
    h = h + b_ref[...]
    h = jnp.maximum(h, 0.0)
    mean = jnp.mean(h, axis=-1, keepdims=True)
    var = jnp.mean((h - mean) ** 2, axis=-1, keepdims=True)
    h = (h - mean) * lax.rsqrt(var + 1e-5)
    out_ref[...] = h * g_ref[...] + beta_ref[...]


_BLK = 2048


def _tc_dense(emb, Wp, b, gamma, beta):
    grid = (BATCH // _BLK,)
    return pl.pallas_call(
        _dense_body,
        grid=grid,
        in_specs=[
            pl.BlockSpec((_BLK, PAD_DIM), lambda i: (i, 0)),
            pl.BlockSpec((PAD_DIM, HIDDEN), lambda i: (0, 0)),
            pl.BlockSpec((1, HIDDEN), lambda i: (0, 0)),
            pl.BlockSpec((1, HIDDEN), lambda i: (0, 0)),
            pl.BlockSpec((1, HIDDEN), lambda i: (0, 0)),
        ],
        out_specs=pl.BlockSpec((_BLK, HIDDEN), lambda i: (i, 0)),
        out_shape=jax.ShapeDtypeStruct((BATCH, HIDDEN), jnp.float32),
    )(emb, Wp, b.reshape(1, HIDDEN), gamma.reshape(1, HIDDEN),
      beta.reshape(1, HIDDEN))


def kernel(user_ids, table, W, b, gamma, beta):
    ids = user_ids.astype(jnp.int32)
    tablep = jnp.pad(table, ((0, 0), (0, PAD_DIM - EMB_DIM)))
    Wp = jnp.pad(W, ((0, PAD_DIM - EMB_DIM), (0, 0)))
    emb = _sc_gather(tablep, ids)
    return _tc_dense(emb, Wp, b, gamma, beta)


# speedup vs baseline: 1.7026x; 1.5375x over previous
import functools

import jax
import jax.numpy as jnp
from jax import lax
from jax.experimental import pallas as pl
from jax.experimental.pallas import tpu as pltpu
from jax.experimental.pallas import tpu_sc as plsc

BATCH = 16384
EMB_DIM = 32
HIDDEN = 128
NUM_USERS = 1000000

_U = 4096
_NBLK = -(-NUM_USERS // _U)
_T2_ROWS = _NBLK * (_U // 4)

_NC = 2
_NS = 16
_NW = _NC * _NS
_B_PER_W = BATCH // _NW
_CHUNK = 128
_NCHUNK = _B_PER_W // _CHUNK
_GRP = 16


def _repack_body(in_ref, out_ref):
    for q in range(4):
        out_ref[:, pl.ds(32 * q, 32)] = jnp.transpose(
            in_ref[:, pl.ds((_U // 4) * q, _U // 4)])


def _tc_repack(tablet):
    return pl.pallas_call(
        _repack_body,
        grid=(_NBLK,),
        in_specs=[pl.BlockSpec((EMB_DIM, _U), lambda i: (0, i))],
        out_specs=pl.BlockSpec((_U // 4, 128), lambda i: (i, 0)),
        out_shape=jax.ShapeDtypeStruct((_T2_ROWS, 128), jnp.float32),
    )(tablet)


def _gather_body(t2_hbm, rows_hbm, idxf_hbm, out_hbm,
                 ridx_v, idx_f, rows_v, emb_v, sem):
    wid = lax.axis_index("s") * _NC + lax.axis_index("c")
    base = wid * _B_PER_W
    pltpu.sync_copy(rows_hbm.at[wid], ridx_v)
    pltpu.sync_copy(idxf_hbm.at[wid], idx_f)
    lane_iota = lax.iota(jnp.int32, 16)
    for h in range(2):
        copies = [
            pltpu.async_copy(
                t2_hbm.at[ridx_v.at[2 * h + j]],
                rows_v.at[pl.ds(j * _CHUNK, _CHUNK)],
                sem,
            )
            for j in range(_NCHUNK // 2)
        ]
        for c in copies:
            c.wait()

        def group(g, carry):
            r0 = g * _GRP
            vec = idx_f[pl.ds(h * 256 + r0, _GRP)]
            for j in range(_GRP):
                off = ((vec[j] >> 10) & 3) * 32
                row = jnp.full((16,), r0 + j, jnp.int32)
                erow = jnp.full((16,), h * 256 + r0 + j, jnp.int32)
                for k in range(2):
                    lanes = off + k * 16 + lane_iota
                    val = plsc.load_gather(rows_v, [row, lanes])
                    plsc.store_scatter(emb_v, [erow, k * 16 + lane_iota],
                                       val)
            return carry

        lax.fori_loop(0, 256 // _GRP, group, 0)
    pltpu.sync_copy(emb_v, out_hbm.at[pl.ds(base, _B_PER_W)])


@functools.partial(jax.jit, static_argnames=())
def _sc_gather(t2, rows, idxf):
    mesh = plsc.VectorSubcoreMesh(core_axis_name="c", subcore_axis_name="s")
    k = pl.kernel(
        _gather_body,
        out_type=jax.ShapeDtypeStruct((BATCH, EMB_DIM), jnp.float32),
        mesh=mesh,
        scratch_types=[
            pltpu.VMEM((_NCHUNK, _CHUNK), jnp.int32),
            pltpu.VMEM((_B_PER_W,), jnp.int32),
            pltpu.VMEM((_B_PER_W // 2, 128), jnp.float32),
            pltpu.VMEM((_B_PER_W, EMB_DIM), jnp.float32),
            pltpu.SemaphoreType.DMA,
        ],
        compiler_params=pltpu.CompilerParams(needs_layout_passes=False),
    )
    return k(t2, rows.reshape(_NW, _NCHUNK, _CHUNK),
             idxf.reshape(_NW, _B_PER_W))


def _dense_body(emb_ref, w_ref, b_ref, g_ref, beta_ref, out_ref):
    h = jnp.dot(emb_ref[...], w_ref[...], preferred_element_type=jnp.float32)
    h = h + b_ref[...]
    h = jnp.maximum(h, 0.0)
    mean = jnp.mean(h, axis=-1, keepdims=True)
    var = jnp.mean((h - mean) ** 2, axis=-1, keepdims=True)
    h = (h - mean) * lax.rsqrt(var + 1e-5)
    out_ref[...] = h * g_ref[...] + beta_ref[...]


_BLK = 2048


def _tc_dense(emb, W, b, gamma, beta):
    grid = (BATCH // _BLK,)
    return pl.pallas_call(
        _dense_body,
        grid=grid,
        in_specs=[
            pl.BlockSpec((_BLK, EMB_DIM), lambda i: (i, 0)),
            pl.BlockSpec((EMB_DIM, HIDDEN), lambda i: (0, 0)),
            pl.BlockSpec((1, HIDDEN), lambda i: (0, 0)),
            pl.BlockSpec((1, HIDDEN), lambda i: (0, 0)),
            pl.BlockSpec((1, HIDDEN), lambda i: (0, 0)),
        ],
        out_specs=pl.BlockSpec((_BLK, HIDDEN), lambda i: (i, 0)),
        out_shape=jax.ShapeDtypeStruct((BATCH, HIDDEN), jnp.float32),
    )(emb, W, b.reshape(1, HIDDEN), gamma.reshape(1, HIDDEN),
      beta.reshape(1, HIDDEN))


def kernel(user_ids, table, W, b, gamma, beta):
    ids = user_ids.astype(jnp.int32)
    t2 = _tc_repack(table.T)
    t2rows = ((ids >> 12) << 10) | (ids & 1023)
    emb = _sc_gather(t2, t2rows, ids)
    return _tc_dense(emb, W, b, gamma, beta)


# speedup vs baseline: 1.9075x; 1.1203x over previous
import functools

import jax
import jax.numpy as jnp
from jax import lax
from jax.experimental import pallas as pl
from jax.experimental.pallas import tpu as pltpu
from jax.experimental.pallas import tpu_sc as plsc

BATCH = 16384
EMB_DIM = 32
HIDDEN = 128
NUM_USERS = 1000000

_U = 4096
_NBLK = -(-NUM_USERS // _U)
_T2_ROWS = _NBLK * (_U // 4)

_NC = 2
_NS = 16
_NW = _NC * _NS
_B_PER_W = BATCH // _NW
_CHUNK = 128
_NCHUNK = _B_PER_W // _CHUNK
_GRP = 16


def _repack_body(in_ref, e_ref, out_ref):
    acc = None
    for q in range(4):
        t = lax.dot_general(
            in_ref[:, pl.ds((_U // 4) * q, _U // 4)], e_ref[q],
            dimension_numbers=(((0,), (0,)), ((), ())),
            preferred_element_type=jnp.float32,
        )
        acc = t if acc is None else acc + t
    out_ref[...] = acc


def _tc_repack(tablet, esel):
    return pl.pallas_call(
        _repack_body,
        grid=(_NBLK,),
        in_specs=[
            pl.BlockSpec((EMB_DIM, _U), lambda i: (0, i)),
            pl.BlockSpec((4, EMB_DIM, 128), lambda i: (0, 0, 0)),
        ],
        out_specs=pl.BlockSpec((_U // 4, 128), lambda i: (i, 0)),
        out_shape=jax.ShapeDtypeStruct((_T2_ROWS, 128), jnp.float32),
        compiler_params=pltpu.CompilerParams(
            fuse_transposed_lhs_in_matmul=True),
    )(tablet, esel)


def _gather_body(t2_hbm, rows_hbm, idxf_hbm, out_hbm,
                 ridx_v, idx_f, rows_v, emb_v, sem):
    wid = lax.axis_index("s") * _NC + lax.axis_index("c")
    base = wid * _B_PER_W
    pltpu.sync_copy(rows_hbm.at[wid], ridx_v)
    pltpu.sync_copy(idxf_hbm.at[wid], idx_f)
    lane_iota = lax.iota(jnp.int32, 16)
    for h in range(2):
        copies = [
            pltpu.async_copy(
                t2_hbm.at[ridx_v.at[2 * h + j]],
                rows_v.at[pl.ds(j * _CHUNK, _CHUNK)],
                sem,
            )
            for j in range(_NCHUNK // 2)
        ]
        for c in copies:
            c.wait()

        def group(g, carry):
            r0 = g * _GRP
            vec = idx_f[pl.ds(h * 256 + r0, _GRP)]
            for j in range(_GRP):
                off = ((vec[j] >> 10) & 3) * 32
                row = jnp.full((16,), r0 + j, jnp.int32)
                erow = jnp.full((16,), h * 256 + r0 + j, jnp.int32)
                for k in range(2):
                    lanes = off + k * 16 + lane_iota
                    val = plsc.load_gather(rows_v, [row, lanes])
                    plsc.store_scatter(emb_v, [erow, k * 16 + lane_iota],
                                       val)
            return carry

        lax.fori_loop(0, 256 // _GRP, group, 0)
    pltpu.sync_copy(emb_v, out_hbm.at[pl.ds(base, _B_PER_W)])


@functools.partial(jax.jit, static_argnames=())
def _sc_gather(t2, rows, idxf):
    mesh = plsc.VectorSubcoreMesh(core_axis_name="c", subcore_axis_name="s")
    k = pl.kernel(
        _gather_body,
        out_type=jax.ShapeDtypeStruct((BATCH, EMB_DIM), jnp.float32),
        mesh=mesh,
        scratch_types=[
            pltpu.VMEM((_NCHUNK, _CHUNK), jnp.int32),
            pltpu.VMEM((_B_PER_W,), jnp.int32),
            pltpu.VMEM((_B_PER_W // 2, 128), jnp.float32),
            pltpu.VMEM((_B_PER_W, EMB_DIM), jnp.float32),
            pltpu.SemaphoreType.DMA,
        ],
        compiler_params=pltpu.CompilerParams(needs_layout_passes=False),
    )
    return k(t2, rows.reshape(_NW, _NCHUNK, _CHUNK),
             idxf.reshape(_NW, _B_PER_W))


def _dense_body(emb_ref, w_ref, b_ref, g_ref, beta_ref, out_ref):
    h = jnp.dot(emb_ref[...], w_ref[...], preferred_element_type=jnp.float32)
    h = h + b_ref[...]
    h = jnp.maximum(h, 0.0)
    mean = jnp.mean(h, axis=-1, keepdims=True)
    var = jnp.mean((h - mean) ** 2, axis=-1, keepdims=True)
    h = (h - mean) * lax.rsqrt(var + 1e-5)
    out_ref[...] = h * g_ref[...] + beta_ref[...]


_BLK = 2048


def _tc_dense(emb, W, b, gamma, beta):
    grid = (BATCH // _BLK,)
    return pl.pallas_call(
        _dense_body,
        grid=grid,
        in_specs=[
            pl.BlockSpec((_BLK, EMB_DIM), lambda i: (i, 0)),
            pl.BlockSpec((EMB_DIM, HIDDEN), lambda i: (0, 0)),
            pl.BlockSpec((1, HIDDEN), lambda i: (0, 0)),
            pl.BlockSpec((1, HIDDEN), lambda i: (0, 0)),
            pl.BlockSpec((1, HIDDEN), lambda i: (0, 0)),
        ],
        out_specs=pl.BlockSpec((_BLK, HIDDEN), lambda i: (i, 0)),
        out_shape=jax.ShapeDtypeStruct((BATCH, HIDDEN), jnp.float32),
    )(emb, W, b.reshape(1, HIDDEN), gamma.reshape(1, HIDDEN),
      beta.reshape(1, HIDDEN))


def kernel(user_ids, table, W, b, gamma, beta):
    ids = user_ids.astype(jnp.int32)
    d_iota = jnp.arange(EMB_DIM)
    q_iota = jnp.arange(4)
    lane = jnp.arange(128)
    esel = (lane[None, None, :] ==
            (32 * q_iota[:, None, None] + d_iota[None, :, None])
            ).astype(jnp.float32)
    t2 = _tc_repack(table.T, esel)
    t2rows = ((ids >> 12) << 10) | (ids & 1023)
    emb = _sc_gather(t2, t2rows, ids)
    return _tc_dense(emb, W, b, gamma, beta)


# speedup vs baseline: 2.3049x; 1.2084x over previous
import functools

import jax
import jax.numpy as jnp
from jax import lax
from jax.experimental import pallas as pl
from jax.experimental.pallas import tpu as pltpu
from jax.experimental.pallas import tpu_sc as plsc

BATCH = 16384
EMB_DIM = 32
HIDDEN = 128
NUM_USERS = 1000000

_U = 4096
_NBLK = -(-NUM_USERS // _U)
_T2_ROWS = _NBLK * (_U // 4)

_NC = 2
_NS = 16
_NW = _NC * _NS
_B_PER_W = BATCH // _NW
_CHUNK = 128
_NCHUNK = _B_PER_W // _CHUNK
_GRP = 16


def _repack_body(in_ref, eye_ref, out_ref):
    l4 = jnp.concatenate(
        [in_ref[:, pl.ds((_U // 4) * q, _U // 4)] for q in range(4)],
        axis=0)
    out_ref[...] = lax.dot_general(
        l4, eye_ref[...],
        dimension_numbers=(((0,), (0,)), ((), ())),
        preferred_element_type=jnp.float32,
    )


def _tc_repack(tablet, eye):
    return pl.pallas_call(
        _repack_body,
        grid=(_NBLK,),
        in_specs=[
            pl.BlockSpec((EMB_DIM, _U), lambda i: (0, i)),
            pl.BlockSpec((128, 128), lambda i: (0, 0)),
        ],
        out_specs=pl.BlockSpec((_U // 4, 128), lambda i: (i, 0)),
        out_shape=jax.ShapeDtypeStruct((_T2_ROWS, 128), jnp.float32),
        compiler_params=pltpu.CompilerParams(
            fuse_transposed_lhs_in_matmul=True),
    )(tablet, eye)


def _gather_body(t2_hbm, rows_hbm, idxf_hbm, out_hbm,
                 ridx_v, idx_f, rows_v, emb_v, sem):
    wid = lax.axis_index("s") * _NC + lax.axis_index("c")
    base = wid * _B_PER_W
    pltpu.sync_copy(rows_hbm.at[wid], ridx_v)
    pltpu.sync_copy(idxf_hbm.at[wid], idx_f)
    lane_iota = lax.iota(jnp.int32, 16)
    for h in range(2):
        copies = [
            pltpu.async_copy(
                t2_hbm.at[ridx_v.at[2 * h + j]],
                rows_v.at[pl.ds(j * _CHUNK, _CHUNK)],
                sem,
            )
            for j in range(_NCHUNK // 2)
        ]
        for c in copies:
            c.wait()

        def group(g, carry):
            r0 = g * _GRP
            vec = idx_f[pl.ds(h * 256 + r0, _GRP)]
            for j in range(_GRP):
                off = ((vec[j] >> 10) & 3) * 32
                row = jnp.full((16,), r0 + j, jnp.int32)
                erow = jnp.full((16,), h * 256 + r0 + j, jnp.int32)
                for k in range(2):
                    lanes = off + k * 16 + lane_iota
                    val = plsc.load_gather(rows_v, [row, lanes])
                    plsc.store_scatter(emb_v, [erow, k * 16 + lane_iota],
                                       val)
            return carry

        lax.fori_loop(0, 256 // _GRP, group, 0)
    pltpu.sync_copy(emb_v, out_hbm.at[pl.ds(base, _B_PER_W)])


@functools.partial(jax.jit, static_argnames=())
def _sc_gather(t2, rows, idxf):
    mesh = plsc.VectorSubcoreMesh(core_axis_name="c", subcore_axis_name="s")
    k = pl.kernel(
        _gather_body,
        out_type=jax.ShapeDtypeStruct((BATCH, EMB_DIM), jnp.float32),
        mesh=mesh,
        scratch_types=[
            pltpu.VMEM((_NCHUNK, _CHUNK), jnp.int32),
            pltpu.VMEM((_B_PER_W,), jnp.int32),
            pltpu.VMEM((_B_PER_W // 2, 128), jnp.float32),
            pltpu.VMEM((_B_PER_W, EMB_DIM), jnp.float32),
            pltpu.SemaphoreType.DMA,
        ],
        compiler_params=pltpu.CompilerParams(needs_layout_passes=False),
    )
    return k(t2, rows.reshape(_NW, _NCHUNK, _CHUNK),
             idxf.reshape(_NW, _B_PER_W))


def _dense_body(emb_ref, w_ref, b_ref, g_ref, beta_ref, out_ref):
    h = jnp.dot(emb_ref[...], w_ref[...], preferred_element_type=jnp.float32)
    h = h + b_ref[...]
    h = jnp.maximum(h, 0.0)
    mean = jnp.mean(h, axis=-1, keepdims=True)
    var = jnp.mean((h - mean) ** 2, axis=-1, keepdims=True)
    h = (h - mean) * lax.rsqrt(var + 1e-5)
    out_ref[...] = h * g_ref[...] + beta_ref[...]


_BLK = 2048


def _tc_dense(emb, W, b, gamma, beta):
    grid = (BATCH // _BLK,)
    return pl.pallas_call(
        _dense_body,
        grid=grid,
        in_specs=[
            pl.BlockSpec((_BLK, EMB_DIM), lambda i: (i, 0)),
            pl.BlockSpec((EMB_DIM, HIDDEN), lambda i: (0, 0)),
            pl.BlockSpec((1, HIDDEN), lambda i: (0, 0)),
            pl.BlockSpec((1, HIDDEN), lambda i: (0, 0)),
            pl.BlockSpec((1, HIDDEN), lambda i: (0, 0)),
        ],
        out_specs=pl.BlockSpec((_BLK, HIDDEN), lambda i: (i, 0)),
        out_shape=jax.ShapeDtypeStruct((BATCH, HIDDEN), jnp.float32),
    )(emb, W, b.reshape(1, HIDDEN), gamma.reshape(1, HIDDEN),
      beta.reshape(1, HIDDEN))


def kernel(user_ids, table, W, b, gamma, beta):
    ids = user_ids.astype(jnp.int32)
    t2 = _tc_repack(table.T, jnp.eye(128, dtype=jnp.float32))
    t2rows = ((ids >> 12) << 10) | (ids & 1023)
    emb = _sc_gather(t2, t2rows, ids)
    return _tc_dense(emb, W, b, gamma, beta)


# speedup vs baseline: 3.1120x; 1.3502x over previous
import functools

import jax
import jax.numpy as jnp
from jax import lax
from jax.experimental import pallas as pl
from jax.experimental.pallas import tpu as pltpu
from jax.experimental.pallas import tpu_sc as plsc

BATCH = 16384
EMB_DIM = 32
HIDDEN = 128
NUM_USERS = 1000000

_U = 8192
_NBLK = -(-NUM_USERS // _U)
_T2_ROWS = _NBLK * (_U // 4)

_NC = 2
_NS = 16
_NW = _NC * _NS
_B_PER_W = BATCH // _NW
_CHUNK = 128
_NCHUNK = _B_PER_W // _CHUNK
_GRP = 16


def _repack_body(in_ref, eye_ref, out_ref):
    l4 = jnp.concatenate(
        [in_ref[:, pl.ds((_U // 4) * q, _U // 4)] for q in range(4)],
        axis=0)
    out_ref[...] = lax.dot_general(
        l4, eye_ref[...],
        dimension_numbers=(((0,), (0,)), ((), ())),
        preferred_element_type=jnp.float32,
    )


def _tc_repack(tablet, eye):
    return pl.pallas_call(
        _repack_body,
        grid=(_NBLK,),
        in_specs=[
            pl.BlockSpec((EMB_DIM, _U), lambda i: (0, i)),
            pl.BlockSpec((128, 128), lambda i: (0, 0)),
        ],
        out_specs=pl.BlockSpec((_U // 4, 128), lambda i: (i, 0)),
        out_shape=jax.ShapeDtypeStruct((_T2_ROWS, 128), jnp.float32),
        compiler_params=pltpu.CompilerParams(
            fuse_transposed_lhs_in_matmul=True),
    )(tablet, eye)


def _gather_body(t2_hbm, rows_hbm, idxf_hbm, out_hbm,
                 ridx_v, idx_f, rows_v, emb_v, sem):
    wid = lax.axis_index("s") * _NC + lax.axis_index("c")
    base = wid * _B_PER_W
    pltpu.sync_copy(rows_hbm.at[wid], ridx_v)
    pltpu.sync_copy(idxf_hbm.at[wid], idx_f)
    lane_iota = lax.iota(jnp.int32, 16)
    for h in range(2):
        copies = [
            pltpu.async_copy(
                t2_hbm.at[ridx_v.at[2 * h + j]],
                rows_v.at[pl.ds(j * _CHUNK, _CHUNK)],
                sem,
            )
            for j in range(_NCHUNK // 2)
        ]
        for c in copies:
            c.wait()

        def group(g, carry):
            r0 = g * _GRP
            vec = idx_f[pl.ds(h * 256 + r0, _GRP)]
            for j in range(_GRP):
                off = ((vec[j] >> 11) & 3) * 32
                row = jnp.full((16,), r0 + j, jnp.int32)
                erow = jnp.full((16,), h * 256 + r0 + j, jnp.int32)
                for k in range(2):
                    lanes = off + k * 16 + lane_iota
                    val = plsc.load_gather(rows_v, [row, lanes])
                    plsc.store_scatter(emb_v, [erow, k * 16 + lane_iota],
                                       val)
            return carry

        lax.fori_loop(0, 256 // _GRP, group, 0)
    pltpu.sync_copy(emb_v, out_hbm.at[pl.ds(base, _B_PER_W)])


@functools.partial(jax.jit, static_argnames=())
def _sc_gather(t2, rows, idxf):
    mesh = plsc.VectorSubcoreMesh(core_axis_name="c", subcore_axis_name="s")
    k = pl.kernel(
        _gather_body,
        out_type=jax.ShapeDtypeStruct((BATCH, EMB_DIM), jnp.float32),
        mesh=mesh,
        scratch_types=[
            pltpu.VMEM((_NCHUNK, _CHUNK), jnp.int32),
            pltpu.VMEM((_B_PER_W,), jnp.int32),
            pltpu.VMEM((_B_PER_W // 2, 128), jnp.float32),
            pltpu.VMEM((_B_PER_W, EMB_DIM), jnp.float32),
            pltpu.SemaphoreType.DMA,
        ],
        compiler_params=pltpu.CompilerParams(needs_layout_passes=False),
    )
    return k(t2, rows.reshape(_NW, _NCHUNK, _CHUNK),
             idxf.reshape(_NW, _B_PER_W))


def _dense_body(emb_ref, w_ref, b_ref, g_ref, beta_ref, out_ref):
    h = jnp.dot(emb_ref[...], w_ref[...], preferred_element_type=jnp.float32)
    h = h + b_ref[...]
    h = jnp.maximum(h, 0.0)
    mean = jnp.mean(h, axis=-1, keepdims=True)
    var = jnp.mean((h - mean) ** 2, axis=-1, keepdims=True)
    h = (h - mean) * lax.rsqrt(var + 1e-5)
    out_ref[...] = h * g_ref[...] + beta_ref[...]


_BLK = 2048


def _tc_dense(emb, W, b, gamma, beta):
    grid = (BATCH // _BLK,)
    return pl.pallas_call(
        _dense_body,
        grid=grid,
        in_specs=[
            pl.BlockSpec((_BLK, EMB_DIM), lambda i: (i, 0)),
            pl.BlockSpec((EMB_DIM, HIDDEN), lambda i: (0, 0)),
            pl.BlockSpec((1, HIDDEN), lambda i: (0, 0)),
            pl.BlockSpec((1, HIDDEN), lambda i: (0, 0)),
            pl.BlockSpec((1, HIDDEN), lambda i: (0, 0)),
        ],
        out_specs=pl.BlockSpec((_BLK, HIDDEN), lambda i: (i, 0)),
        out_shape=jax.ShapeDtypeStruct((BATCH, HIDDEN), jnp.float32),
    )(emb, W, b.reshape(1, HIDDEN), gamma.reshape(1, HIDDEN),
      beta.reshape(1, HIDDEN))


def kernel(user_ids, table, W, b, gamma, beta):
    ids = user_ids.astype(jnp.int32)
    t2 = _tc_repack(table.T, jnp.eye(128, dtype=jnp.float32))
    t2rows = ((ids >> 13) << 11) | (ids & 2047)
    emb = _sc_gather(t2, t2rows, ids)
    return _tc_dense(emb, W, b, gamma, beta)


# speedup vs baseline: 3.9569x; 1.2715x over previous
import functools

import jax
import jax.numpy as jnp
from jax import lax
from jax.experimental import pallas as pl
from jax.experimental.pallas import tpu as pltpu
from jax.experimental.pallas import tpu_sc as plsc

BATCH = 16384
EMB_DIM = 32
HIDDEN = 128
NUM_USERS = 1000000

_U = 16384
_NBLK = -(-NUM_USERS // _U)
_T2_ROWS = _NBLK * (_U // 4)

_NC = 2
_NS = 16
_NW = _NC * _NS
_B_PER_W = BATCH // _NW
_CHUNK = 128
_NCHUNK = _B_PER_W // _CHUNK
_GRP = 16


def _repack_body(in_ref, eye_ref, out_ref):
    l4 = jnp.concatenate(
        [in_ref[:, pl.ds((_U // 4) * q, _U // 4)] for q in range(4)],
        axis=0)
    out_ref[...] = lax.dot_general(
        l4, eye_ref[...],
        dimension_numbers=(((0,), (0,)), ((), ())),
        preferred_element_type=jnp.float32,
    )


def _tc_repack(tablet, eye):
    return pl.pallas_call(
        _repack_body,
        grid=(_NBLK,),
        in_specs=[
            pl.BlockSpec((EMB_DIM, _U), lambda i: (0, i)),
            pl.BlockSpec((128, 128), lambda i: (0, 0)),
        ],
        out_specs=pl.BlockSpec((_U // 4, 128), lambda i: (i, 0)),
        out_shape=jax.ShapeDtypeStruct((_T2_ROWS, 128), jnp.float32),
        compiler_params=pltpu.CompilerParams(
            fuse_transposed_lhs_in_matmul=True),
    )(tablet, eye)


def _gather_body(t2_hbm, rows_hbm, idxf_hbm, out_hbm,
                 ridx_v, idx_f, rows_v, emb_v, sem):
    wid = lax.axis_index("s") * _NC + lax.axis_index("c")
    base = wid * _B_PER_W
    pltpu.sync_copy(rows_hbm.at[wid], ridx_v)
    pltpu.sync_copy(idxf_hbm.at[wid], idx_f)
    lane_iota = lax.iota(jnp.int32, 16)
    for h in range(2):
        copies = [
            pltpu.async_copy(
                t2_hbm.at[ridx_v.at[2 * h + j]],
                rows_v.at[pl.ds(j * _CHUNK, _CHUNK)],
                sem,
            )
            for j in range(_NCHUNK // 2)
        ]
        for c in copies:
            c.wait()

        def group(g, carry):
            r0 = g * _GRP
            vec = idx_f[pl.ds(h * 256 + r0, _GRP)]
            for j in range(_GRP):
                off = ((vec[j] >> 12) & 3) * 32
                row = jnp.full((16,), r0 + j, jnp.int32)
                erow = jnp.full((16,), h * 256 + r0 + j, jnp.int32)
                for k in range(2):
                    lanes = off + k * 16 + lane_iota
                    val = plsc.load_gather(rows_v, [row, lanes])
                    plsc.store_scatter(emb_v, [erow, k * 16 + lane_iota],
                                       val)
            return carry

        lax.fori_loop(0, 256 // _GRP, group, 0)
    pltpu.sync_copy(emb_v, out_hbm.at[pl.ds(base, _B_PER_W)])


@functools.partial(jax.jit, static_argnames=())
def _sc_gather(t2, rows, idxf):
    mesh = plsc.VectorSubcoreMesh(core_axis_name="c", subcore_axis_name="s")
    k = pl.kernel(
        _gather_body,
        out_type=jax.ShapeDtypeStruct((BATCH, EMB_DIM), jnp.float32),
        mesh=mesh,
        scratch_types=[
            pltpu.VMEM((_NCHUNK, _CHUNK), jnp.int32),
            pltpu.VMEM((_B_PER_W,), jnp.int32),
            pltpu.VMEM((_B_PER_W // 2, 128), jnp.float32),
            pltpu.VMEM((_B_PER_W, EMB_DIM), jnp.float32),
            pltpu.SemaphoreType.DMA,
        ],
        compiler_params=pltpu.CompilerParams(needs_layout_passes=False),
    )
    return k(t2, rows.reshape(_NW, _NCHUNK, _CHUNK),
             idxf.reshape(_NW, _B_PER_W))


def _dense_body(emb_ref, w_ref, b_ref, g_ref, beta_ref, out_ref):
    h = jnp.dot(emb_ref[...], w_ref[...], preferred_element_type=jnp.float32)
    h = h + b_ref[...]
    h = jnp.maximum(h, 0.0)
    mean = jnp.mean(h, axis=-1, keepdims=True)
    var = jnp.mean((h - mean) ** 2, axis=-1, keepdims=True)
    h = (h - mean) * lax.rsqrt(var + 1e-5)
    out_ref[...] = h * g_ref[...] + beta_ref[...]


_BLK = 2048


def _tc_dense(emb, W, b, gamma, beta):
    grid = (BATCH // _BLK,)
    return pl.pallas_call(
        _dense_body,
        grid=grid,
        in_specs=[
            pl.BlockSpec((_BLK, EMB_DIM), lambda i: (i, 0)),
            pl.BlockSpec((EMB_DIM, HIDDEN), lambda i: (0, 0)),
            pl.BlockSpec((1, HIDDEN), lambda i: (0, 0)),
            pl.BlockSpec((1, HIDDEN), lambda i: (0, 0)),
            pl.BlockSpec((1, HIDDEN), lambda i: (0, 0)),
        ],
        out_specs=pl.BlockSpec((_BLK, HIDDEN), lambda i: (i, 0)),
        out_shape=jax.ShapeDtypeStruct((BATCH, HIDDEN), jnp.float32),
    )(emb, W, b.reshape(1, HIDDEN), gamma.reshape(1, HIDDEN),
      beta.reshape(1, HIDDEN))


def kernel(user_ids, table, W, b, gamma, beta):
    ids = user_ids.astype(jnp.int32)
    t2 = _tc_repack(table.T, jnp.eye(128, dtype=jnp.float32))
    t2rows = ((ids >> 14) << 12) | (ids & 4095)
    emb = _sc_gather(t2, t2rows, ids)
    return _tc_dense(emb, W, b, gamma, beta)


# speedup vs baseline: 4.4267x; 1.1187x over previous
import functools

import jax
import jax.numpy as jnp
from jax import lax
from jax.experimental import pallas as pl
from jax.experimental.pallas import tpu as pltpu
from jax.experimental.pallas import tpu_sc as plsc

BATCH = 16384
EMB_DIM = 32
HIDDEN = 128
NUM_USERS = 1000000

_U = 32768
_NBLK = -(-NUM_USERS // _U)
_T2_ROWS = _NBLK * (_U // 4)

_NC = 2
_NS = 16
_NW = _NC * _NS
_B_PER_W = BATCH // _NW
_CHUNK = 128
_NCHUNK = _B_PER_W // _CHUNK
_GRP = 16


def _repack_body(in_ref, eye_ref, out_ref):
    l4 = jnp.concatenate(
        [in_ref[:, pl.ds((_U // 4) * q, _U // 4)] for q in range(4)],
        axis=0)
    out_ref[...] = lax.dot_general(
        l4, eye_ref[...],
        dimension_numbers=(((0,), (0,)), ((), ())),
        preferred_element_type=jnp.float32,
    )


def _tc_repack(tablet, eye):
    return pl.pallas_call(
        _repack_body,
        grid=(_NBLK,),
        in_specs=[
            pl.BlockSpec((EMB_DIM, _U), lambda i: (0, i)),
            pl.BlockSpec((128, 128), lambda i: (0, 0)),
        ],
        out_specs=pl.BlockSpec((_U // 4, 128), lambda i: (i, 0)),
        out_shape=jax.ShapeDtypeStruct((_T2_ROWS, 128), jnp.float32),
        compiler_params=pltpu.CompilerParams(
            fuse_transposed_lhs_in_matmul=True),
    )(tablet, eye)


def _gather_body(t2_hbm, rows_hbm, idxf_hbm, out_hbm,
                 ridx_v, idx_f, rows_v, emb_v, sem):
    wid = lax.axis_index("s") * _NC + lax.axis_index("c")
    base = wid * _B_PER_W
    pltpu.sync_copy(rows_hbm.at[wid], ridx_v)
    pltpu.sync_copy(idxf_hbm.at[wid], idx_f)
    lane_iota = lax.iota(jnp.int32, 16)
    for h in range(2):
        copies = [
            pltpu.async_copy(
                t2_hbm.at[ridx_v.at[2 * h + j]],
                rows_v.at[pl.ds(j * _CHUNK, _CHUNK)],
                sem,
            )
            for j in range(_NCHUNK // 2)
        ]
        for c in copies:
            c.wait()

        def group(g, carry):
            r0 = g * _GRP
            vec = idx_f[pl.ds(h * 256 + r0, _GRP)]
            for j in range(_GRP):
                off = ((vec[j] >> 13) & 3) * 32
                row = jnp.full((16,), r0 + j, jnp.int32)
                erow = jnp.full((16,), h * 256 + r0 + j, jnp.int32)
                for k in range(2):
                    lanes = off + k * 16 + lane_iota
                    val = plsc.load_gather(rows_v, [row, lanes])
                    plsc.store_scatter(emb_v, [erow, k * 16 + lane_iota],
                                       val)
            return carry

        lax.fori_loop(0, 256 // _GRP, group, 0)
    pltpu.sync_copy(emb_v, out_hbm.at[pl.ds(base, _B_PER_W)])


@functools.partial(jax.jit, static_argnames=())
def _sc_gather(t2, rows, idxf):
    mesh = plsc.VectorSubcoreMesh(core_axis_name="c", subcore_axis_name="s")
    k = pl.kernel(
        _gather_body,
        out_type=jax.ShapeDtypeStruct((BATCH, EMB_DIM), jnp.float32),
        mesh=mesh,
        scratch_types=[
            pltpu.VMEM((_NCHUNK, _CHUNK), jnp.int32),
            pltpu.VMEM((_B_PER_W,), jnp.int32),
            pltpu.VMEM((_B_PER_W // 2, 128), jnp.float32),
            pltpu.VMEM((_B_PER_W, EMB_DIM), jnp.float32),
            pltpu.SemaphoreType.DMA,
        ],
        compiler_params=pltpu.CompilerParams(needs_layout_passes=False),
    )
    return k(t2, rows.reshape(_NW, _NCHUNK, _CHUNK),
             idxf.reshape(_NW, _B_PER_W))


def _dense_body(emb_ref, w_ref, b_ref, g_ref, beta_ref, out_ref):
    h = jnp.dot(emb_ref[...], w_ref[...], preferred_element_type=jnp.float32)
    h = h + b_ref[...]
    h = jnp.maximum(h, 0.0)
    mean = jnp.mean(h, axis=-1, keepdims=True)
    var = jnp.mean((h - mean) ** 2, axis=-1, keepdims=True)
    h = (h - mean) * lax.rsqrt(var + 1e-5)
    out_ref[...] = h * g_ref[...] + beta_ref[...]


_BLK = 2048


def _tc_dense(emb, W, b, gamma, beta):
    grid = (BATCH // _BLK,)
    return pl.pallas_call(
        _dense_body,
        grid=grid,
        in_specs=[
            pl.BlockSpec((_BLK, EMB_DIM), lambda i: (i, 0)),
            pl.BlockSpec((EMB_DIM, HIDDEN), lambda i: (0, 0)),
            pl.BlockSpec((1, HIDDEN), lambda i: (0, 0)),
            pl.BlockSpec((1, HIDDEN), lambda i: (0, 0)),
            pl.BlockSpec((1, HIDDEN), lambda i: (0, 0)),
        ],
        out_specs=pl.BlockSpec((_BLK, HIDDEN), lambda i: (i, 0)),
        out_shape=jax.ShapeDtypeStruct((BATCH, HIDDEN), jnp.float32),
    )(emb, W, b.reshape(1, HIDDEN), gamma.reshape(1, HIDDEN),
      beta.reshape(1, HIDDEN))


def kernel(user_ids, table, W, b, gamma, beta):
    ids = user_ids.astype(jnp.int32)
    t2 = _tc_repack(table.T, jnp.eye(128, dtype=jnp.float32))
    t2rows = ((ids >> 15) << 13) | (ids & 8191)
    emb = _sc_gather(t2, t2rows, ids)
    return _tc_dense(emb, W, b, gamma, beta)


# speedup vs baseline: 4.4550x; 1.0064x over previous
import functools

import jax
import jax.numpy as jnp
from jax import lax
from jax.experimental import pallas as pl
from jax.experimental.pallas import tpu as pltpu
from jax.experimental.pallas import tpu_sc as plsc

BATCH = 16384
EMB_DIM = 32
HIDDEN = 128
NUM_USERS = 1000000

_U = 65536
_NBLK = -(-NUM_USERS // _U)
_T2_ROWS = _NBLK * (_U // 4)

_NC = 2
_NS = 16
_NW = _NC * _NS
_B_PER_W = BATCH // _NW
_CHUNK = 128
_NCHUNK = _B_PER_W // _CHUNK
_GRP = 16


def _repack_body(in_ref, eye_ref, out_ref):
    l4 = jnp.concatenate(
        [in_ref[:, pl.ds((_U // 4) * q, _U // 4)] for q in range(4)],
        axis=0)
    out_ref[...] = lax.dot_general(
        l4, eye_ref[...],
        dimension_numbers=(((0,), (0,)), ((), ())),
        preferred_element_type=jnp.float32,
    )


def _tc_repack(tablet, eye):
    return pl.pallas_call(
        _repack_body,
        grid=(_NBLK,),
        in_specs=[
            pl.BlockSpec((EMB_DIM, _U), lambda i: (0, i)),
            pl.BlockSpec((128, 128), lambda i: (0, 0)),
        ],
        out_specs=pl.BlockSpec((_U // 4, 128), lambda i: (i, 0)),
        out_shape=jax.ShapeDtypeStruct((_T2_ROWS, 128), jnp.float32),
        compiler_params=pltpu.CompilerParams(
            fuse_transposed_lhs_in_matmul=True),
    )(tablet, eye)


def _gather_body(t2_hbm, rows_hbm, idxf_hbm, out_hbm,
                 ridx_v, idx_f, rows_v, emb_v, sem):
    wid = lax.axis_index("s") * _NC + lax.axis_index("c")
    base = wid * _B_PER_W
    pltpu.sync_copy(rows_hbm.at[wid], ridx_v)
    pltpu.sync_copy(idxf_hbm.at[wid], idx_f)
    lane_iota = lax.iota(jnp.int32, 16)
    for h in range(2):
        copies = [
            pltpu.async_copy(
                t2_hbm.at[ridx_v.at[2 * h + j]],
                rows_v.at[pl.ds(j * _CHUNK, _CHUNK)],
                sem,
            )
            for j in range(_NCHUNK // 2)
        ]
        for c in copies:
            c.wait()

        def group(g, carry):
            r0 = g * _GRP
            vec = idx_f[pl.ds(h * 256 + r0, _GRP)]
            for j in range(_GRP):
                off = ((vec[j] >> 14) & 3) * 32
                row = jnp.full((16,), r0 + j, jnp.int32)
                erow = jnp.full((16,), h * 256 + r0 + j, jnp.int32)
                for k in range(2):
                    lanes = off + k * 16 + lane_iota
                    val = plsc.load_gather(rows_v, [row, lanes])
                    plsc.store_scatter(emb_v, [erow, k * 16 + lane_iota],
                                       val)
            return carry

        lax.fori_loop(0, 256 // _GRP, group, 0)
    pltpu.sync_copy(emb_v, out_hbm.at[pl.ds(base, _B_PER_W)])


@functools.partial(jax.jit, static_argnames=())
def _sc_gather(t2, rows, idxf):
    mesh = plsc.VectorSubcoreMesh(core_axis_name="c", subcore_axis_name="s")
    k = pl.kernel(
        _gather_body,
        out_type=jax.ShapeDtypeStruct((BATCH, EMB_DIM), jnp.float32),
        mesh=mesh,
        scratch_types=[
            pltpu.VMEM((_NCHUNK, _CHUNK), jnp.int32),
            pltpu.VMEM((_B_PER_W,), jnp.int32),
            pltpu.VMEM((_B_PER_W // 2, 128), jnp.float32),
            pltpu.VMEM((_B_PER_W, EMB_DIM), jnp.float32),
            pltpu.SemaphoreType.DMA,
        ],
        compiler_params=pltpu.CompilerParams(needs_layout_passes=False),
    )
    return k(t2, rows.reshape(_NW, _NCHUNK, _CHUNK),
             idxf.reshape(_NW, _B_PER_W))


def _dense_body(emb_ref, w_ref, b_ref, g_ref, beta_ref, out_ref):
    h = jnp.dot(emb_ref[...], w_ref[...], preferred_element_type=jnp.float32)
    h = h + b_ref[...]
    h = jnp.maximum(h, 0.0)
    mean = jnp.mean(h, axis=-1, keepdims=True)
    var = jnp.mean((h - mean) ** 2, axis=-1, keepdims=True)
    h = (h - mean) * lax.rsqrt(var + 1e-5)
    out_ref[...] = h * g_ref[...] + beta_ref[...]


_BLK = 2048


def _tc_dense(emb, W, b, gamma, beta):
    grid = (BATCH // _BLK,)
    return pl.pallas_call(
        _dense_body,
        grid=grid,
        in_specs=[
            pl.BlockSpec((_BLK, EMB_DIM), lambda i: (i, 0)),
            pl.BlockSpec((EMB_DIM, HIDDEN), lambda i: (0, 0)),
            pl.BlockSpec((1, HIDDEN), lambda i: (0, 0)),
            pl.BlockSpec((1, HIDDEN), lambda i: (0, 0)),
            pl.BlockSpec((1, HIDDEN), lambda i: (0, 0)),
        ],
        out_specs=pl.BlockSpec((_BLK, HIDDEN), lambda i: (i, 0)),
        out_shape=jax.ShapeDtypeStruct((BATCH, HIDDEN), jnp.float32),
    )(emb, W, b.reshape(1, HIDDEN), gamma.reshape(1, HIDDEN),
      beta.reshape(1, HIDDEN))


def kernel(user_ids, table, W, b, gamma, beta):
    ids = user_ids.astype(jnp.int32)
    t2 = _tc_repack(table.T, jnp.eye(128, dtype=jnp.float32))
    t2rows = ((ids >> 16) << 14) | (ids & 16383)
    emb = _sc_gather(t2, t2rows, ids)
    return _tc_dense(emb, W, b, gamma, beta)
